# baseline (speedup 1.0000x reference)
import jax
import jax.numpy as jnp
from jax.experimental import pallas as pl


def _identity_body(x_ref, o_ref):
    o_ref[...] = x_ref[...]


def _sage_conv(x, edge_index, Wl, bl, Wr):
    src = edge_index[0]
    dst = edge_index[1]
    msg = jnp.take(x, src, axis=0)
    agg = jax.ops.segment_sum(msg, dst, num_segments=x.shape[0])
    deg = jax.ops.segment_sum(jnp.ones((edge_index.shape[1],), x.dtype), dst,
                              num_segments=x.shape[0])
    mean = agg / jnp.maximum(deg, 1.0)[:, None]
    return mean @ Wl.T + bl + x @ Wr.T


def kernel(x, edge_index, W_emb, b_emb, Wl0, bl0, Wr0, Wl1, bl1, Wr1, Wl2, bl2, Wr2):
    h = x @ W_emb.T + b_emb
    h = jax.nn.relu(_sage_conv(h, edge_index, Wl0, bl0, Wr0))
    h = jax.nn.relu(_sage_conv(h, edge_index, Wl1, bl1, Wr1))
    h = _sage_conv(h, edge_index, Wl2, bl2, Wr2)
    return pl.pallas_call(
        _identity_body,
        out_shape=jax.ShapeDtypeStruct(h.shape, h.dtype),
    )(h)


# speedup vs baseline: 7.8770x; 7.8770x over previous
import functools

import jax
import jax.numpy as jnp
from jax import lax
from jax.experimental import pallas as pl
from jax.experimental.pallas import tpu as pltpu
from jax.experimental.pallas import tpu_sc as plsc

N = 10000
E = 320000
D = 128
NC = 2
NS = 16
DH = D // NC
C = 80
TILE_EDGES = E // NS
CHUNKS = TILE_EDGES // C
NPAD = 10240
ROWS_PER_TILE = NPAD // NS
DEGW = 16
BR = 400


def _sc_agg_body(hw_hbm, src_hbm, dst_hbm, p_hbm, deg_hbm,
                 src_idx, dst_idx, rows0, rows1, ones_v, zeros_v,
                 acc, dacc, sem0, sem1):
    cid = lax.axis_index("c")
    sid = lax.axis_index("s")

    @pl.loop(0, C)
    def _fill(i):
        ones_v[i, :] = jnp.ones((DEGW,), jnp.float32)
        zeros_v[i, :] = jnp.zeros((DEGW,), jnp.float32)
        for j in range(DH // 16):
            rows0[i, pl.ds(j * 16, 16)] = jnp.zeros((16,), jnp.float32)

    base = sid * ROWS_PER_TILE

    @pl.loop(0, ROWS_PER_TILE // C)
    def _zero(k):
        pltpu.sync_copy(rows0, acc.at[pl.ds(base + k * C, C)])
        pltpu.sync_copy(zeros_v, dacc.at[pl.ds(base + k * C, C)])

    pltpu.sync_copy(src_hbm.at[sid], src_idx)
    pltpu.sync_copy(dst_hbm.at[sid], dst_idx)

    plsc.subcore_barrier()

    hw_half = hw_hbm.at[cid]

    def start_gather(j, buf, sem):
        pltpu.async_copy(hw_half.at[src_idx.at[j]], buf, sem)

    def wait_gather(buf, sem):
        pltpu.make_async_copy(hw_half.at[src_idx.at[0]], buf, sem).wait()

    def process(j, buf, sem):
        wait_gather(buf, sem)
        pltpu.sync_copy(buf, acc.at[dst_idx.at[j]], add=True)

        @pl.when(cid == 0)
        def _deg():
            pltpu.sync_copy(ones_v, dacc.at[dst_idx.at[j]], add=True)

    start_gather(0, rows0, sem0)
    start_gather(1, rows1, sem1)

    @pl.loop(0, CHUNKS - 2, step=2)
    def _main(j):
        process(j, rows0, sem0)
        start_gather(j + 2, rows0, sem0)
        process(j + 1, rows1, sem1)
        start_gather(j + 3, rows1, sem1)

    process(CHUNKS - 2, rows0, sem0)
    process(CHUNKS - 1, rows1, sem1)

    plsc.subcore_barrier()
    pltpu.sync_copy(acc.at[pl.ds(base, ROWS_PER_TILE)],
                    p_hbm.at[cid, pl.ds(base, ROWS_PER_TILE)])

    @pl.when(cid == 0)
    def _deg_out():
        pltpu.sync_copy(dacc.at[pl.ds(base, ROWS_PER_TILE)],
                        deg_hbm.at[pl.ds(base, ROWS_PER_TILE)])


@functools.cache
def _get_sc_agg():
  return pl.kernel(
    _sc_agg_body,
    out_type=(jax.ShapeDtypeStruct((NC, NPAD, DH), jnp.float32),
              jax.ShapeDtypeStruct((NPAD, DEGW), jnp.float32)),
    mesh=plsc.VectorSubcoreMesh(core_axis_name="c", subcore_axis_name="s",
                                num_cores=NC, num_subcores=NS),
    compiler_params=pltpu.CompilerParams(use_tc_tiling_on_sc=False),
    scratch_types=[
        pltpu.VMEM((CHUNKS, C), jnp.int32),
        pltpu.VMEM((CHUNKS, C), jnp.int32),
        pltpu.VMEM((C, DH), jnp.float32),
        pltpu.VMEM((C, DH), jnp.float32),
        pltpu.VMEM((C, DEGW), jnp.float32),
        pltpu.VMEM((C, DEGW), jnp.float32),
        pltpu.VMEM_SHARED((NPAD, DH), jnp.float32),
        pltpu.VMEM_SHARED((NPAD, DEGW), jnp.float32),
        pltpu.SemaphoreType.DMA,
        pltpu.SemaphoreType.DMA,
    ],
  )


def _dot_t(a, w):
    return lax.dot_general(a, w, (((1,), (1,)), ((), ())),
                           preferred_element_type=jnp.float32,
                           precision=lax.Precision.HIGHEST)


def _split_store(hw_ref, t):
    hw_ref[0] = t[:, :DH]
    hw_ref[1] = t[:, DH:]


def _emb_body(x_ref, wemb_ref, bemb_ref, wl_ref, h_ref, hw_ref):
    h = _dot_t(x_ref[...], wemb_ref[...]) + bemb_ref[...]
    h_ref[...] = h
    _split_store(hw_ref, _dot_t(h, wl_ref[...]))


_emb = pl.pallas_call(
    _emb_body,
    grid=(N // BR,),
    in_specs=[pl.BlockSpec((BR, D), lambda i: (i, 0)),
              pl.BlockSpec((D, D), lambda i: (0, 0)),
              pl.BlockSpec((1, D), lambda i: (0, 0)),
              pl.BlockSpec((D, D), lambda i: (0, 0))],
    out_specs=[pl.BlockSpec((BR, D), lambda i: (i, 0)),
               pl.BlockSpec((NC, BR, DH), lambda i: (0, i, 0))],
    out_shape=[jax.ShapeDtypeStruct((N, D), jnp.float32),
               jax.ShapeDtypeStruct((NC, N, DH), jnp.float32)],
)


def _make_combine(with_relu, with_next):
    def body(*refs):
        if with_next:
            (p_ref, d_ref, h_ref, wr_ref, bl_ref, wl_ref,
             out_ref, hw_ref) = refs
        else:
            p_ref, d_ref, h_ref, wr_ref, bl_ref, out_ref = refs
        deg = d_ref[:, 0]
        scale = 1.0 / jnp.maximum(deg, 1.0)
        agg = jnp.concatenate([p_ref[0], p_ref[1]], axis=1)
        t = agg * scale[:, None]
        t = t + _dot_t(h_ref[...], wr_ref[...]) + bl_ref[...]
        if with_relu:
            t = jnp.maximum(t, 0.0)
        out_ref[...] = t
        if with_next:
            _split_store(hw_ref, _dot_t(t, wl_ref[...]))

    in_specs = [pl.BlockSpec((NC, BR, DH), lambda i: (0, i, 0)),
                pl.BlockSpec((BR, DEGW), lambda i: (i, 0)),
                pl.BlockSpec((BR, D), lambda i: (i, 0)),
                pl.BlockSpec((D, D), lambda i: (0, 0)),
                pl.BlockSpec((1, D), lambda i: (0, 0))]
    out_specs = [pl.BlockSpec((BR, D), lambda i: (i, 0))]
    out_shape = [jax.ShapeDtypeStruct((N, D), jnp.float32)]
    if with_next:
        in_specs.append(pl.BlockSpec((D, D), lambda i: (0, 0)))
        out_specs = out_specs + [pl.BlockSpec((NC, BR, DH), lambda i: (0, i, 0))]
        out_shape = out_shape + [jax.ShapeDtypeStruct((NC, N, DH), jnp.float32)]
    return pl.pallas_call(
        body, grid=(N // BR,),
        in_specs=in_specs, out_specs=out_specs, out_shape=out_shape)


_combine_next = _make_combine(True, True)
_combine_last = _make_combine(False, False)


def kernel(x, edge_index, W_emb, b_emb, Wl0, bl0, Wr0, Wl1, bl1, Wr1, Wl2, bl2, Wr2):
    src2 = edge_index[0].reshape(NS, CHUNKS, C)
    dst2 = edge_index[1].reshape(NS, CHUNKS, C)
    b_emb2 = b_emb.reshape(1, D)
    bl0_2 = bl0.reshape(1, D)
    bl1_2 = bl1.reshape(1, D)
    bl2_2 = bl2.reshape(1, D)

    sc_agg = _get_sc_agg()
    h0, hw0 = _emb(x, W_emb, b_emb2, Wl0)
    p, degp = sc_agg(hw0, src2, dst2)
    h1, hw1 = _combine_next(p, degp, h0, Wr0, bl0_2, Wl1)
    p, _ = sc_agg(hw1, src2, dst2)
    h2, hw2 = _combine_next(p, degp, h1, Wr1, bl1_2, Wl2)
    p, _ = sc_agg(hw2, src2, dst2)
    out, = _combine_last(p, degp, h2, Wr2, bl2_2)
    return out


# speedup vs baseline: 10.2108x; 1.2963x over previous
import functools

import jax
import jax.numpy as jnp
from jax import lax
from jax.experimental import pallas as pl
from jax.experimental.pallas import tpu as pltpu
from jax.experimental.pallas import tpu_sc as plsc

N = 10000
E = 320000
D = 128
NC = 2
NS = 16
DH = D // NC
C = 100
NB = 4
TILE_EDGES = E // NS
CHUNKS = TILE_EDGES // C
NPAD = 10240
ROWS_PER_TILE = NPAD // NS
ZC = 80
DEGW = 16
BR = 400


def _make_sc_body(with_deg):
  def body(*refs):
    if with_deg:
        (hw_hbm, src_hbm, dst_hbm, p_hbm, deg_hbm,
         src_idx, dst_idx, rows, zbuf, ones_v, zeros_v,
         acc, dacc, gsem, ssem, isem, dsem) = refs
    else:
        (hw_hbm, src_hbm, dst_hbm, p_hbm,
         src_idx, dst_idx, rows, zbuf,
         acc, gsem, ssem, isem) = refs
    cid = lax.axis_index("c")
    sid = lax.axis_index("s")

    pltpu.async_copy(src_hbm.at[sid], src_idx, isem)
    pltpu.async_copy(dst_hbm.at[sid], dst_idx, isem)

    @pl.loop(0, ZC)
    def _fillz(i):
        for j in range(DH // 16):
            zbuf[i, pl.ds(j * 16, 16)] = jnp.zeros((16,), jnp.float32)

    if with_deg:
        @pl.loop(0, C)
        def _fill1(i):
            ones_v[i, :] = jnp.ones((DEGW,), jnp.float32)

        @pl.loop(0, ZC)
        def _fill0(i):
            zeros_v[i, :] = jnp.zeros((DEGW,), jnp.float32)

    base = sid * ROWS_PER_TILE

    @pl.loop(0, ROWS_PER_TILE // ZC)
    def _zero(k):
        pltpu.sync_copy(zbuf, acc.at[pl.ds(base + k * ZC, ZC)])
        if with_deg:
            pltpu.sync_copy(zeros_v, dacc.at[pl.ds(base + k * ZC, ZC)])

    pltpu.make_async_copy(src_hbm.at[sid], src_idx, isem).wait()
    pltpu.make_async_copy(dst_hbm.at[sid], dst_idx, isem).wait()

    plsc.subcore_barrier()

    hw_half = hw_hbm.at[cid]

    def start_gather(j, b):
        pltpu.async_copy(hw_half.at[src_idx.at[j]], rows.at[b], gsem.at[b])

    def wait_gather(b):
        pltpu.make_async_copy(hw_half.at[src_idx.at[0]], rows.at[b],
                              gsem.at[b]).wait()

    def start_scatter(j, b):
        pltpu.async_copy(rows.at[b], acc.at[dst_idx.at[j]], ssem.at[b],
                         add=True)
        if with_deg:
            @pl.when(cid == 0)
            def _deg():
                pltpu.async_copy(ones_v, dacc.at[dst_idx.at[j]], dsem.at[b],
                                 add=True)

    def wait_scatter(b):
        pltpu.make_async_copy(rows.at[b], acc.at[dst_idx.at[0]],
                              ssem.at[b]).wait()
        if with_deg:
            @pl.when(cid == 0)
            def _deg():
                pltpu.make_async_copy(ones_v, dacc.at[dst_idx.at[0]],
                                      dsem.at[b]).wait()

    for b in range(NB):
        start_gather(b, b)

    @pl.loop(0, CHUNKS - NB, step=NB)
    def _main(j):
        for b in range(NB):
            wait_gather(b)
            start_scatter(j + b, b)
        for b in range(NB):
            wait_scatter(b)
            start_gather(j + NB + b, b)

    for b in range(NB):
        wait_gather(b)
        start_scatter(CHUNKS - NB + b, b)
    for b in range(NB):
        wait_scatter(b)

    plsc.subcore_barrier()
    pltpu.sync_copy(acc.at[pl.ds(base, ROWS_PER_TILE)],
                    p_hbm.at[cid, pl.ds(base, ROWS_PER_TILE)])

    if with_deg:
        @pl.when(cid == 0)
        def _deg_out():
            pltpu.sync_copy(dacc.at[pl.ds(base, ROWS_PER_TILE)],
                            deg_hbm.at[pl.ds(base, ROWS_PER_TILE)])

  return body


@functools.cache
def _get_sc_agg(with_deg):
  out_type = [jax.ShapeDtypeStruct((NC, NPAD, DH), jnp.float32)]
  scratch = [
      pltpu.VMEM((CHUNKS, C), jnp.int32),
      pltpu.VMEM((CHUNKS, C), jnp.int32),
      pltpu.VMEM((NB, C, DH), jnp.float32),
      pltpu.VMEM((ZC, DH), jnp.float32),
  ]
  if with_deg:
      out_type.append(jax.ShapeDtypeStruct((NPAD, DEGW), jnp.float32))
      scratch.append(pltpu.VMEM((C, DEGW), jnp.float32))
      scratch.append(pltpu.VMEM((ZC, DEGW), jnp.float32))
  scratch.append(pltpu.VMEM_SHARED((NPAD, DH), jnp.float32))
  if with_deg:
      scratch.append(pltpu.VMEM_SHARED((NPAD, DEGW), jnp.float32))
  scratch.append(pltpu.SemaphoreType.DMA((NB,)))
  scratch.append(pltpu.SemaphoreType.DMA((NB,)))
  scratch.append(pltpu.SemaphoreType.DMA)
  if with_deg:
      scratch.append(pltpu.SemaphoreType.DMA((NB,)))
  return pl.kernel(
      _make_sc_body(with_deg),
      out_type=tuple(out_type) if with_deg else out_type[0],
      mesh=plsc.VectorSubcoreMesh(core_axis_name="c", subcore_axis_name="s",
                                  num_cores=NC, num_subcores=NS),
      compiler_params=pltpu.CompilerParams(use_tc_tiling_on_sc=False),
      scratch_types=scratch,
  )


def _dot_t(a, w):
    return lax.dot_general(a, w, (((1,), (1,)), ((), ())),
                           preferred_element_type=jnp.float32,
                           precision=lax.Precision.HIGHEST)


def _split_store(hw_ref, t):
    hw_ref[0] = t[:, :DH]
    hw_ref[1] = t[:, DH:]


def _emb_body(x_ref, wemb_ref, bemb_ref, wl_ref, h_ref, hw_ref):
    h = _dot_t(x_ref[...], wemb_ref[...]) + bemb_ref[...]
    h_ref[...] = h
    _split_store(hw_ref, _dot_t(h, wl_ref[...]))


_emb = pl.pallas_call(
    _emb_body,
    grid=(N // BR,),
    in_specs=[pl.BlockSpec((BR, D), lambda i: (i, 0)),
              pl.BlockSpec((D, D), lambda i: (0, 0)),
              pl.BlockSpec((1, D), lambda i: (0, 0)),
              pl.BlockSpec((D, D), lambda i: (0, 0))],
    out_specs=[pl.BlockSpec((BR, D), lambda i: (i, 0)),
               pl.BlockSpec((NC, BR, DH), lambda i: (0, i, 0))],
    out_shape=[jax.ShapeDtypeStruct((N, D), jnp.float32),
               jax.ShapeDtypeStruct((NC, N, DH), jnp.float32)],
)


def _make_combine(with_relu, with_next):
    def body(*refs):
        if with_next:
            (p_ref, d_ref, h_ref, wr_ref, bl_ref, wl_ref,
             out_ref, hw_ref) = refs
        else:
            p_ref, d_ref, h_ref, wr_ref, bl_ref, out_ref = refs
        deg = d_ref[:, 0]
        scale = 1.0 / jnp.maximum(deg, 1.0)
        agg = jnp.concatenate([p_ref[0], p_ref[1]], axis=1)
        t = agg * scale[:, None]
        t = t + _dot_t(h_ref[...], wr_ref[...]) + bl_ref[...]
        if with_relu:
            t = jnp.maximum(t, 0.0)
        out_ref[...] = t
        if with_next:
            _split_store(hw_ref, _dot_t(t, wl_ref[...]))

    in_specs = [pl.BlockSpec((NC, BR, DH), lambda i: (0, i, 0)),
                pl.BlockSpec((BR, DEGW), lambda i: (i, 0)),
                pl.BlockSpec((BR, D), lambda i: (i, 0)),
                pl.BlockSpec((D, D), lambda i: (0, 0)),
                pl.BlockSpec((1, D), lambda i: (0, 0))]
    out_specs = [pl.BlockSpec((BR, D), lambda i: (i, 0))]
    out_shape = [jax.ShapeDtypeStruct((N, D), jnp.float32)]
    if with_next:
        in_specs.append(pl.BlockSpec((D, D), lambda i: (0, 0)))
        out_specs = out_specs + [pl.BlockSpec((NC, BR, DH), lambda i: (0, i, 0))]
        out_shape = out_shape + [jax.ShapeDtypeStruct((NC, N, DH), jnp.float32)]
    return pl.pallas_call(
        body, grid=(N // BR,),
        in_specs=in_specs, out_specs=out_specs, out_shape=out_shape)


_combine_next = _make_combine(True, True)
_combine_last = _make_combine(False, False)


def kernel(x, edge_index, W_emb, b_emb, Wl0, bl0, Wr0, Wl1, bl1, Wr1, Wl2, bl2, Wr2):
    src2 = edge_index[0].reshape(NS, CHUNKS, C)
    dst2 = edge_index[1].reshape(NS, CHUNKS, C)
    b_emb2 = b_emb.reshape(1, D)
    bl0_2 = bl0.reshape(1, D)
    bl1_2 = bl1.reshape(1, D)
    bl2_2 = bl2.reshape(1, D)

    sc_deg = _get_sc_agg(True)
    sc_plain = _get_sc_agg(False)
    h0, hw0 = _emb(x, W_emb, b_emb2, Wl0)
    p, degp = sc_deg(hw0, src2, dst2)
    h1, hw1 = _combine_next(p, degp, h0, Wr0, bl0_2, Wl1)
    p = sc_plain(hw1, src2, dst2)
    h2, hw2 = _combine_next(p, degp, h1, Wr1, bl1_2, Wl2)
    p = sc_plain(hw2, src2, dst2)
    out, = _combine_last(p, degp, h2, Wr2, bl2_2)
    return out


# speedup vs baseline: 11.3522x; 1.1118x over previous
import functools

import jax
import jax.numpy as jnp
from jax import lax
from jax.experimental import pallas as pl
from jax.experimental.pallas import tpu as pltpu
from jax.experimental.pallas import tpu_sc as plsc

N = 10000
E = 320000
D = 128
NC = 2
NS = 16
DH = D // NC
C = 100
NB = 4
TILE_EDGES = E // NS
CHUNKS = TILE_EDGES // C
NPAD = 10240
ROWS_PER_TILE = NPAD // NS
ZC = 80
DEGW = 16
BR = 1000


def _make_sc_body(with_deg):
  def body(*refs):
    if with_deg:
        (hw_hbm, src_hbm, dst_hbm, p_hbm, deg_hbm,
         src_idx, dst_idx, rows, zbuf, ones_v, zeros_v,
         acc, dacc, gsem, ssem, isem, dsem) = refs
    else:
        (hw_hbm, src_hbm, dst_hbm, p_hbm,
         src_idx, dst_idx, rows, zbuf,
         acc, gsem, ssem, isem) = refs
    cid = lax.axis_index("c")
    sid = lax.axis_index("s")

    pltpu.async_copy(src_hbm.at[sid], src_idx, isem)
    pltpu.async_copy(dst_hbm.at[sid], dst_idx, isem)

    @pl.loop(0, ZC)
    def _fillz(i):
        for j in range(DH // 16):
            zbuf[i, pl.ds(j * 16, 16)] = jnp.zeros((16,), jnp.float32)

    if with_deg:
        @pl.loop(0, C)
        def _fill1(i):
            ones_v[i, :] = jnp.ones((DEGW,), jnp.float32)

        @pl.loop(0, ZC)
        def _fill0(i):
            zeros_v[i, :] = jnp.zeros((DEGW,), jnp.float32)

    base = sid * ROWS_PER_TILE

    @pl.loop(0, ROWS_PER_TILE // ZC)
    def _zero(k):
        pltpu.sync_copy(zbuf, acc.at[pl.ds(base + k * ZC, ZC)])
        if with_deg:
            pltpu.sync_copy(zeros_v, dacc.at[pl.ds(base + k * ZC, ZC)])

    pltpu.make_async_copy(src_hbm.at[sid], src_idx, isem).wait()
    pltpu.make_async_copy(dst_hbm.at[sid], dst_idx, isem).wait()

    plsc.subcore_barrier()

    hw_half = hw_hbm.at[cid]

    def start_gather(j, b):
        pltpu.async_copy(hw_half.at[src_idx.at[j]], rows.at[b], gsem.at[b])

    def wait_gather(b):
        pltpu.make_async_copy(hw_half.at[src_idx.at[0]], rows.at[b],
                              gsem.at[b]).wait()

    def start_scatter(j, b):
        pltpu.async_copy(rows.at[b], acc.at[dst_idx.at[j]], ssem.at[b],
                         add=True)
        if with_deg:
            @pl.when(cid == 0)
            def _deg():
                pltpu.async_copy(ones_v, dacc.at[dst_idx.at[j]], dsem.at[b],
                                 add=True)

    def wait_scatter(b):
        pltpu.make_async_copy(rows.at[b], acc.at[dst_idx.at[0]],
                              ssem.at[b]).wait()
        if with_deg:
            @pl.when(cid == 0)
            def _deg():
                pltpu.make_async_copy(ones_v, dacc.at[dst_idx.at[0]],
                                      dsem.at[b]).wait()

    for b in range(NB):
        start_gather(b, b)

    @pl.loop(0, CHUNKS - NB, step=NB)
    def _main(j):
        for b in range(NB):
            wait_gather(b)
            start_scatter(j + b, b)
        for b in range(NB):
            wait_scatter(b)
            start_gather(j + NB + b, b)

    for b in range(NB):
        wait_gather(b)
        start_scatter(CHUNKS - NB + b, b)
    for b in range(NB):
        wait_scatter(b)

    plsc.subcore_barrier()
    pltpu.sync_copy(acc.at[pl.ds(base, ROWS_PER_TILE)],
                    p_hbm.at[cid, pl.ds(base, ROWS_PER_TILE)])

    if with_deg:
        @pl.when(cid == 0)
        def _deg_out():
            pltpu.sync_copy(dacc.at[pl.ds(base, ROWS_PER_TILE)],
                            deg_hbm.at[pl.ds(base, ROWS_PER_TILE)])

  return body


@functools.cache
def _get_sc_agg(with_deg):
  out_type = [jax.ShapeDtypeStruct((NC, NPAD, DH), jnp.float32)]
  scratch = [
      pltpu.VMEM((CHUNKS, C), jnp.int32),
      pltpu.VMEM((CHUNKS, C), jnp.int32),
      pltpu.VMEM((NB, C, DH), jnp.float32),
      pltpu.VMEM((ZC, DH), jnp.float32),
  ]
  if with_deg:
      out_type.append(jax.ShapeDtypeStruct((NPAD, DEGW), jnp.float32))
      scratch.append(pltpu.VMEM((C, DEGW), jnp.float32))
      scratch.append(pltpu.VMEM((ZC, DEGW), jnp.float32))
  scratch.append(pltpu.VMEM_SHARED((NPAD, DH), jnp.float32))
  if with_deg:
      scratch.append(pltpu.VMEM_SHARED((NPAD, DEGW), jnp.float32))
  scratch.append(pltpu.SemaphoreType.DMA((NB,)))
  scratch.append(pltpu.SemaphoreType.DMA((NB,)))
  scratch.append(pltpu.SemaphoreType.DMA)
  if with_deg:
      scratch.append(pltpu.SemaphoreType.DMA((NB,)))
  return pl.kernel(
      _make_sc_body(with_deg),
      out_type=tuple(out_type) if with_deg else out_type[0],
      mesh=plsc.VectorSubcoreMesh(core_axis_name="c", subcore_axis_name="s",
                                  num_cores=NC, num_subcores=NS),
      compiler_params=pltpu.CompilerParams(use_tc_tiling_on_sc=False),
      scratch_types=scratch,
  )


def _dot_t(a, w):
    return lax.dot_general(a, w, (((1,), (1,)), ((), ())),
                           preferred_element_type=jnp.float32)


def _split_store(hw_ref, t):
    hw_ref[0] = t[:, :DH]
    hw_ref[1] = t[:, DH:]


def _emb_body(x_ref, wemb_ref, bemb_ref, wl_ref, h_ref, hw_ref):
    h = _dot_t(x_ref[...], wemb_ref[...]) + bemb_ref[...]
    h_ref[...] = h
    _split_store(hw_ref, _dot_t(h, wl_ref[...]))


_emb = pl.pallas_call(
    _emb_body,
    grid=(N // BR,),
    in_specs=[pl.BlockSpec((BR, D), lambda i: (i, 0)),
              pl.BlockSpec((D, D), lambda i: (0, 0)),
              pl.BlockSpec((1, D), lambda i: (0, 0)),
              pl.BlockSpec((D, D), lambda i: (0, 0))],
    out_specs=[pl.BlockSpec((BR, D), lambda i: (i, 0)),
               pl.BlockSpec((NC, BR, DH), lambda i: (0, i, 0))],
    out_shape=[jax.ShapeDtypeStruct((N, D), jnp.float32),
               jax.ShapeDtypeStruct((NC, N, DH), jnp.float32)],
)


def _make_combine(with_relu, with_next):
    def body(*refs):
        if with_next:
            (p_ref, d_ref, h_ref, wr_ref, bl_ref, wl_ref,
             out_ref, hw_ref) = refs
        else:
            p_ref, d_ref, h_ref, wr_ref, bl_ref, out_ref = refs
        deg = d_ref[:, 0]
        scale = 1.0 / jnp.maximum(deg, 1.0)
        agg = jnp.concatenate([p_ref[0], p_ref[1]], axis=1)
        t = agg * scale[:, None]
        t = t + _dot_t(h_ref[...], wr_ref[...]) + bl_ref[...]
        if with_relu:
            t = jnp.maximum(t, 0.0)
        out_ref[...] = t
        if with_next:
            _split_store(hw_ref, _dot_t(t, wl_ref[...]))

    in_specs = [pl.BlockSpec((NC, BR, DH), lambda i: (0, i, 0)),
                pl.BlockSpec((BR, DEGW), lambda i: (i, 0)),
                pl.BlockSpec((BR, D), lambda i: (i, 0)),
                pl.BlockSpec((D, D), lambda i: (0, 0)),
                pl.BlockSpec((1, D), lambda i: (0, 0))]
    out_specs = [pl.BlockSpec((BR, D), lambda i: (i, 0))]
    out_shape = [jax.ShapeDtypeStruct((N, D), jnp.float32)]
    if with_next:
        in_specs.append(pl.BlockSpec((D, D), lambda i: (0, 0)))
        out_specs = out_specs + [pl.BlockSpec((NC, BR, DH), lambda i: (0, i, 0))]
        out_shape = out_shape + [jax.ShapeDtypeStruct((NC, N, DH), jnp.float32)]
    return pl.pallas_call(
        body, grid=(N // BR,),
        in_specs=in_specs, out_specs=out_specs, out_shape=out_shape)


_combine_next = _make_combine(True, True)
_combine_last = _make_combine(False, False)


def kernel(x, edge_index, W_emb, b_emb, Wl0, bl0, Wr0, Wl1, bl1, Wr1, Wl2, bl2, Wr2):
    src2 = edge_index[0].reshape(NS, CHUNKS, C)
    dst2 = edge_index[1].reshape(NS, CHUNKS, C)
    b_emb2 = b_emb.reshape(1, D)
    bl0_2 = bl0.reshape(1, D)
    bl1_2 = bl1.reshape(1, D)
    bl2_2 = bl2.reshape(1, D)

    sc_deg = _get_sc_agg(True)
    sc_plain = _get_sc_agg(False)
    h0, hw0 = _emb(x, W_emb, b_emb2, Wl0)
    p, degp = sc_deg(hw0, src2, dst2)
    h1, hw1 = _combine_next(p, degp, h0, Wr0, bl0_2, Wl1)
    p = sc_plain(hw1, src2, dst2)
    h2, hw2 = _combine_next(p, degp, h1, Wr1, bl1_2, Wl2)
    p = sc_plain(hw2, src2, dst2)
    out, = _combine_last(p, degp, h2, Wr2, bl2_2)
    return out
